# baseline (speedup 1.0000x reference)
import functools

import jax
import jax.numpy as jnp
from jax import lax
from jax.experimental import pallas as pl
from jax.experimental.pallas import tpu as pltpu
from jax.experimental.pallas import tpu_sc as plsc

N, C, S = 2048, 8, 2048
L = 16
NCORES = 2
NSUB = 16
CLS_PER_CORE = C // NCORES
CHUNKS = S // L
PBLK = 8

P = 512
SPAN_P = P // NSUB
RB = 128
RBC = 256

EPS = 1.1920928955078125e-07
SCALE = float(100.0 / (N * S + EPS))

_i32 = jnp.int32


def _row_argmax(buf, r, iota, neg_inf, zeros_i):
    def chunk_body(_, carry):
        maxv, maxk, k = carry
        v = buf[_i32(r), pl.ds(k * _i32(L), L)]
        m = v > maxv
        return (jnp.where(m, v, maxv), jnp.where(m, k, maxk), k + _i32(1))

    maxv, maxk, _ = lax.fori_loop(0, CHUNKS, chunk_body,
                                  (neg_inf, zeros_i, _i32(0)), unroll=8)
    mval = jnp.max(maxv)
    cand = jnp.where(maxv == mval, maxk * _i32(L) + iota, _i32(S))
    return jnp.min(cand)


def _argmax_body(pred_hbm, lab_hbm, pbuf0, pbuf1, labbuf, psem0, psem1):
    core = lax.axis_index("c")
    sid = lax.axis_index("s")
    iota = lax.iota(jnp.int32, L)
    neg_inf = jnp.full((L,), -jnp.inf, dtype=jnp.float32)
    zeros_i = jnp.zeros((L,), dtype=jnp.int32)
    s0 = sid * _i32(SPAN_P)

    for cl in range(CLS_PER_CORE):
        c = core * _i32(CLS_PER_CORE) + _i32(cl)

        def pstart(blk, buf, sem, c=c):
            base = jnp.minimum(s0 + blk * _i32(PBLK), _i32(N - PBLK))
            pltpu.async_copy(pred_hbm.at[pl.ds(base, PBLK), c], buf, sem)

        def pwait(sem):
            pltpu.make_async_copy(
                pred_hbm.at[pl.ds(_i32(0), PBLK), _i32(0)], pbuf0, sem).wait()

        pstart(_i32(0), pbuf0, psem0)

        def pgrp(g, _, c=c):
            pstart(_i32(2) * g + _i32(1), pbuf1, psem1, c=c)
            pwait(psem0)
            lab_vec = zeros_i
            for r in range(PBLK):
                lab = _row_argmax(pbuf0, r, iota, neg_inf, zeros_i)
                lab_vec = jnp.where(iota == _i32(r), lab, lab_vec)
            pstart(_i32(2) * g + _i32(2), pbuf0, psem0, c=c)
            pwait(psem1)
            for r in range(PBLK):
                lab = _row_argmax(pbuf1, r, iota, neg_inf, zeros_i)
                lab_vec = jnp.where(iota == _i32(PBLK + r), lab, lab_vec)
            labbuf[pl.ds(g * _i32(L), L)] = lab_vec
            return _i32(0)

        lax.fori_loop(_i32(0), _i32(SPAN_P // (2 * PBLK)), pgrp, _i32(0))
        pwait(psem0)
        pltpu.sync_copy(labbuf, lab_hbm.at[c, pl.ds(s0, SPAN_P)])


def _mesh():
    return plsc.VectorSubcoreMesh(core_axis_name="c", subcore_axis_name="s")


def _tc_argmax(pred_ref, lab_ref):
    x = pred_ref[...]
    m = jnp.max(x, axis=-1, keepdims=True)
    idx = lax.broadcasted_iota(jnp.int32, x.shape, 2)
    first = jnp.min(jnp.where(x == m, idx, _i32(S)), axis=-1)
    lab_ref[...] = first


def _tc_count(lab_ref, targ_ref, out_ref, *, w):
    b = pl.program_id(0)

    @pl.when(b == 0)
    def _():
        out_ref[...] = jnp.zeros((C, w), dtype=jnp.int32)

    t = targ_ref[...]
    lab = lab_ref[...]
    eq = (t == lab[None]).astype(jnp.int32)
    out_ref[...] += jnp.sum(eq, axis=0, dtype=jnp.int32)


def _tc_combine(lo_ref, hi_ref, out_ref):
    lo = lo_ref[...].astype(jnp.float32)
    hi = hi_ref[...].astype(jnp.float32)
    tot = (jnp.sum(lo.reshape(NCORES, CLS_PER_CORE, P), axis=-1)
           + jnp.sum(hi.reshape(NCORES, CLS_PER_CORE, N - P), axis=-1))
    out_ref[...] = (tot + EPS) * SCALE


def _count_call(lab, target, col0, w):
    return pl.pallas_call(
        functools.partial(_tc_count, w=w),
        grid=(N // RBC,),
        in_specs=[
            pl.BlockSpec((C, w), lambda b: (_i32(0), _i32(0))),
            pl.BlockSpec((RBC, C, w),
                         lambda b: (b, _i32(0), _i32(col0 // w))),
        ],
        out_specs=pl.BlockSpec((C, w), lambda b: (_i32(0), _i32(0))),
        out_shape=jax.ShapeDtypeStruct((C, w), jnp.int32),
    )(lab, target)


@jax.jit
def _accuracy(pred, target):
    sc_lab = functools.partial(
        pl.kernel,
        out_type=jax.ShapeDtypeStruct((C, P), jnp.int32),
        mesh=_mesh(),
        compiler_params=pltpu.CompilerParams(needs_layout_passes=False),
        cost_estimate=pl.CostEstimate(flops=2_000_000_000,
                                      transcendentals=0,
                                      bytes_accessed=1_000_000_000),
        scratch_types=[
            pltpu.VMEM((PBLK, S), jnp.float32),
            pltpu.VMEM((PBLK, S), jnp.float32),
            pltpu.VMEM((SPAN_P,), jnp.int32),
            pltpu.SemaphoreType.DMA,
            pltpu.SemaphoreType.DMA,
        ],
    )(_argmax_body)(pred)

    tc_lab = pl.pallas_call(
        _tc_argmax,
        grid=((N - P) // RB,),
        in_specs=[pl.BlockSpec((RB, C, S),
                               lambda b: (b + P // RB, _i32(0), _i32(0)))],
        out_specs=pl.BlockSpec((RB, C), lambda b: (b, _i32(0))),
        out_shape=jax.ShapeDtypeStruct((N - P, C), jnp.int32),
    )(pred)

    cnt_hi = _count_call(tc_lab.T, target, P, N - P)
    cnt_lo = _count_call(sc_lab, target, 0, P)

    return pl.pallas_call(
        _tc_combine,
        out_shape=jax.ShapeDtypeStruct((NCORES, CLS_PER_CORE), jnp.float32),
    )(cnt_lo, cnt_hi)


def kernel(pred, target):
    target = target.astype(jnp.int32)
    return _accuracy(pred, target).reshape(C, 1)


# speedup vs baseline: 1.0065x; 1.0065x over previous
import functools

import jax
import jax.numpy as jnp
from jax import lax
from jax.experimental import pallas as pl
from jax.experimental.pallas import tpu as pltpu
from jax.experimental.pallas import tpu_sc as plsc

N, C, S = 2048, 8, 2048
L = 16
NCORES = 2
NSUB = 16
CLS_PER_CORE = C // NCORES
CHUNKS = S // L
PBLK = 8

P = 1024
SPAN_P = P // NSUB
RB = 128
RBC = 256

EPS = 1.1920928955078125e-07
SCALE = float(100.0 / (N * S + EPS))

_i32 = jnp.int32


def _row_argmax(buf, r, iota, neg_inf, zeros_i):
    def chunk_body(_, carry):
        maxv, maxk, k = carry
        v = buf[_i32(r), pl.ds(k * _i32(L), L)]
        m = v > maxv
        return (jnp.where(m, v, maxv), jnp.where(m, k, maxk), k + _i32(1))

    maxv, maxk, _ = lax.fori_loop(0, CHUNKS, chunk_body,
                                  (neg_inf, zeros_i, _i32(0)), unroll=8)
    mval = jnp.max(maxv)
    cand = jnp.where(maxv == mval, maxk * _i32(L) + iota, _i32(S))
    return jnp.min(cand)


def _argmax_body(pred_hbm, lab_hbm, pbuf0, pbuf1, labbuf, psem0, psem1):
    core = lax.axis_index("c")
    sid = lax.axis_index("s")
    iota = lax.iota(jnp.int32, L)
    neg_inf = jnp.full((L,), -jnp.inf, dtype=jnp.float32)
    zeros_i = jnp.zeros((L,), dtype=jnp.int32)
    s0 = sid * _i32(SPAN_P)

    for cl in range(CLS_PER_CORE):
        c = core * _i32(CLS_PER_CORE) + _i32(cl)

        def pstart(blk, buf, sem, c=c):
            base = jnp.minimum(s0 + blk * _i32(PBLK), _i32(N - PBLK))
            pltpu.async_copy(pred_hbm.at[pl.ds(base, PBLK), c], buf, sem)

        def pwait(sem):
            pltpu.make_async_copy(
                pred_hbm.at[pl.ds(_i32(0), PBLK), _i32(0)], pbuf0, sem).wait()

        pstart(_i32(0), pbuf0, psem0)

        def pgrp(g, _, c=c):
            pstart(_i32(2) * g + _i32(1), pbuf1, psem1, c=c)
            pwait(psem0)
            lab_vec = zeros_i
            for r in range(PBLK):
                lab = _row_argmax(pbuf0, r, iota, neg_inf, zeros_i)
                lab_vec = jnp.where(iota == _i32(r), lab, lab_vec)
            pstart(_i32(2) * g + _i32(2), pbuf0, psem0, c=c)
            pwait(psem1)
            for r in range(PBLK):
                lab = _row_argmax(pbuf1, r, iota, neg_inf, zeros_i)
                lab_vec = jnp.where(iota == _i32(PBLK + r), lab, lab_vec)
            labbuf[pl.ds(g * _i32(L), L)] = lab_vec
            return _i32(0)

        lax.fori_loop(_i32(0), _i32(SPAN_P // (2 * PBLK)), pgrp, _i32(0))
        pwait(psem0)
        pltpu.sync_copy(labbuf, lab_hbm.at[c, pl.ds(s0, SPAN_P)])


def _mesh():
    return plsc.VectorSubcoreMesh(core_axis_name="c", subcore_axis_name="s")


def _tc_argmax(pred_ref, lab_ref):
    x = pred_ref[...]
    m = jnp.max(x, axis=-1, keepdims=True)
    idx = lax.broadcasted_iota(jnp.int32, x.shape, 2)
    first = jnp.min(jnp.where(x == m, idx, _i32(S)), axis=-1)
    lab_ref[...] = first


def _tc_count(lab_ref, targ_ref, out_ref, *, w):
    b = pl.program_id(0)

    @pl.when(b == 0)
    def _():
        out_ref[...] = jnp.zeros((C, w), dtype=jnp.int32)

    t = targ_ref[...]
    lab = lab_ref[...]
    eq = (t == lab[None]).astype(jnp.int32)
    out_ref[...] += jnp.sum(eq, axis=0, dtype=jnp.int32)


def _tc_combine(lo_ref, hi_ref, out_ref):
    lo = lo_ref[...].astype(jnp.float32)
    hi = hi_ref[...].astype(jnp.float32)
    tot = (jnp.sum(lo.reshape(NCORES, CLS_PER_CORE, P), axis=-1)
           + jnp.sum(hi.reshape(NCORES, CLS_PER_CORE, N - P), axis=-1))
    out_ref[...] = (tot + EPS) * SCALE


def _count_call(lab, target, col0, w):
    return pl.pallas_call(
        functools.partial(_tc_count, w=w),
        grid=(N // RBC,),
        in_specs=[
            pl.BlockSpec((C, w), lambda b: (_i32(0), _i32(0))),
            pl.BlockSpec((RBC, C, w),
                         lambda b: (b, _i32(0), _i32(col0 // w))),
        ],
        out_specs=pl.BlockSpec((C, w), lambda b: (_i32(0), _i32(0))),
        out_shape=jax.ShapeDtypeStruct((C, w), jnp.int32),
    )(lab, target)


@jax.jit
def _accuracy(pred, target):
    sc_lab = functools.partial(
        pl.kernel,
        out_type=jax.ShapeDtypeStruct((C, P), jnp.int32),
        mesh=_mesh(),
        compiler_params=pltpu.CompilerParams(needs_layout_passes=False),
        cost_estimate=pl.CostEstimate(flops=2_000_000_000,
                                      transcendentals=0,
                                      bytes_accessed=1_000_000_000),
        scratch_types=[
            pltpu.VMEM((PBLK, S), jnp.float32),
            pltpu.VMEM((PBLK, S), jnp.float32),
            pltpu.VMEM((SPAN_P,), jnp.int32),
            pltpu.SemaphoreType.DMA,
            pltpu.SemaphoreType.DMA,
        ],
    )(_argmax_body)(pred)

    tc_lab = pl.pallas_call(
        _tc_argmax,
        grid=((N - P) // RB,),
        in_specs=[pl.BlockSpec((RB, C, S),
                               lambda b: (b + P // RB, _i32(0), _i32(0)))],
        out_specs=pl.BlockSpec((RB, C), lambda b: (b, _i32(0))),
        out_shape=jax.ShapeDtypeStruct((N - P, C), jnp.int32),
    )(pred)

    cnt_hi = _count_call(tc_lab.T, target, P, N - P)
    cnt_lo = _count_call(sc_lab, target, 0, P)

    return pl.pallas_call(
        _tc_combine,
        out_shape=jax.ShapeDtypeStruct((NCORES, CLS_PER_CORE), jnp.float32),
    )(cnt_lo, cnt_hi)


def kernel(pred, target):
    target = target.astype(jnp.int32)
    return _accuracy(pred, target).reshape(C, 1)
